# baseline (speedup 1.0000x reference)
import functools

import jax
import jax.numpy as jnp
from jax.experimental import pallas as pl
from jax.experimental.pallas import tpu as pltpu

_NE = 1024
_DIM = 64
_CC = 0.25
_TB = 1024
_NTOK = 32768
_NB = _NTOK // _TB
_NELEM = float(_NTOK * _DIM)


def _vq_body(x_ref, w_ref, x2_ref, qst_ref, idx_ref, loss_ref):
    x = x_ref[...]
    w = w_ref[...]
    x2 = x2_ref[...]

    s = jax.lax.dot_general(
        x, w, (((1,), (1,)), ((), ())),
        preferred_element_type=jnp.float32,
        precision=jax.lax.Precision.DEFAULT)
    w2 = jnp.sum(w * w, axis=1)
    d = (x2 + w2[None, :]) - 2.0 * s
    dmin = jnp.min(d, axis=1, keepdims=True)
    iota = jax.lax.broadcasted_iota(jnp.int32, (_TB, _NE), 1)
    idx = jnp.min(jnp.where(d == dmin, iota, _NE), axis=1).astype(jnp.int32)

    part = jnp.sum(dmin)

    step = pl.program_id(0)

    @pl.when(step == 0)
    def _init():
        loss_ref[0, 0] = 0.0

    loss_ref[0, 0] += part

    @pl.when(step == _NB - 1)
    def _finalize():
        m = loss_ref[0, 0] / _NELEM
        loss_ref[0, 0] = m + _CC * m

    onehot = (jax.lax.broadcasted_iota(jnp.int32, (_TB, _NE), 1)
              == idx[:, None]).astype(jnp.float32)
    q = jax.lax.dot_general(
        onehot, w, (((1,), (0,)), ((), ())),
        preferred_element_type=jnp.float32,
        precision=jax.lax.Precision.HIGHEST)

    qst_ref[...] = x + (q - x)
    idx_ref[0, 0, :] = idx


@functools.partial(jax.jit, static_argnums=())
def kernel(inputs, weight):
    B, C, D, H, W = inputs.shape
    flat = jnp.transpose(inputs, (0, 2, 3, 4, 1)).reshape(-1, C)
    x2_5d = jnp.sum(jnp.transpose(inputs, (0, 2, 3, 4, 1)) ** 2, axis=4)
    x2 = x2_5d.reshape(-1, 1)

    qst_flat, idx_blocked, loss = pl.pallas_call(
        _vq_body,
        grid=(_NB,),
        in_specs=[
            pl.BlockSpec((_TB, _DIM), lambda i: (i, 0)),
            pl.BlockSpec((_NE, _DIM), lambda i: (0, 0)),
            pl.BlockSpec((_TB, 1), lambda i: (i, 0)),
        ],
        out_specs=[
            pl.BlockSpec((_TB, _DIM), lambda i: (i, 0)),
            pl.BlockSpec((1, 1, _TB), lambda i: (i, 0, 0)),
            pl.BlockSpec((1, 1), lambda i: (0, 0), memory_space=pltpu.SMEM),
        ],
        out_shape=[
            jax.ShapeDtypeStruct((_NTOK, _DIM), jnp.float32),
            jax.ShapeDtypeStruct((_NB, 1, _TB), jnp.int32),
            jax.ShapeDtypeStruct((1, 1), jnp.float32),
        ],
    )(flat, weight, x2)

    quantized_st = jnp.transpose(
        qst_flat.reshape(B, D, H, W, C), (0, 4, 1, 2, 3))
    encoding_indices = idx_blocked.reshape(B, D, H, W)
    return (quantized_st, loss[0, 0], encoding_indices)


# speedup vs baseline: 1.4298x; 1.4298x over previous
import functools

import jax
import jax.numpy as jnp
from jax.experimental import pallas as pl
from jax.experimental.pallas import tpu as pltpu

_NE = 1024
_DIM = 64
_CC = 0.25
_TB = 1024
_NTOK = 32768
_NB = _NTOK // _TB
_NELEM = float(_NTOK * _DIM)


def _vq_body(x_ref, w_ref, x2_ref, qst_ref, idx_ref, loss_ref):
    x = x_ref[...]
    w = w_ref[...]
    x2 = x2_ref[...]

    s = jax.lax.dot_general(
        x, w, (((1,), (1,)), ((), ())),
        preferred_element_type=jnp.float32,
        precision=jax.lax.Precision.DEFAULT)
    w2 = jnp.sum(w * w, axis=1)
    d = (x2 + w2[None, :]) - 2.0 * s
    dmin = jnp.min(d, axis=1, keepdims=True)
    iota = jax.lax.broadcasted_iota(jnp.int32, (_TB, _NE), 1)
    idx = jnp.min(jnp.where(d == dmin, iota, _NE), axis=1).astype(jnp.int32)

    part = jnp.sum(dmin)

    step = pl.program_id(0)

    @pl.when(step == 0)
    def _init():
        loss_ref[0, 0] = 0.0

    loss_ref[0, 0] += part

    @pl.when(step == _NB - 1)
    def _finalize():
        m = loss_ref[0, 0] / _NELEM
        loss_ref[0, 0] = m + _CC * m

    onehot = (jax.lax.broadcasted_iota(jnp.int32, (_TB, _NE), 1)
              == idx[:, None]).astype(jnp.float32)
    w_hi = w.astype(jnp.bfloat16).astype(jnp.float32)
    w_lo = w - w_hi
    q_hi = jax.lax.dot_general(
        onehot, w_hi, (((1,), (0,)), ((), ())),
        preferred_element_type=jnp.float32,
        precision=jax.lax.Precision.DEFAULT)
    q_lo = jax.lax.dot_general(
        onehot, w_lo, (((1,), (0,)), ((), ())),
        preferred_element_type=jnp.float32,
        precision=jax.lax.Precision.DEFAULT)
    q = q_hi + q_lo

    qst_ref[...] = x + (q - x)
    idx_ref[0, 0, :] = idx


@functools.partial(jax.jit, static_argnums=())
def kernel(inputs, weight):
    B, C, D, H, W = inputs.shape
    flat = jnp.transpose(inputs, (0, 2, 3, 4, 1)).reshape(-1, C)
    x2_5d = jnp.sum(jnp.transpose(inputs, (0, 2, 3, 4, 1)) ** 2, axis=4)
    x2 = x2_5d.reshape(-1, 1)

    qst_flat, idx_blocked, loss = pl.pallas_call(
        _vq_body,
        grid=(_NB,),
        in_specs=[
            pl.BlockSpec((_TB, _DIM), lambda i: (i, 0)),
            pl.BlockSpec((_NE, _DIM), lambda i: (0, 0)),
            pl.BlockSpec((_TB, 1), lambda i: (i, 0)),
        ],
        out_specs=[
            pl.BlockSpec((_TB, _DIM), lambda i: (i, 0)),
            pl.BlockSpec((1, 1, _TB), lambda i: (i, 0, 0)),
            pl.BlockSpec((1, 1), lambda i: (0, 0), memory_space=pltpu.SMEM),
        ],
        out_shape=[
            jax.ShapeDtypeStruct((_NTOK, _DIM), jnp.float32),
            jax.ShapeDtypeStruct((_NB, 1, _TB), jnp.int32),
            jax.ShapeDtypeStruct((1, 1), jnp.float32),
        ],
    )(flat, weight, x2)

    quantized_st = jnp.transpose(
        qst_flat.reshape(B, D, H, W, C), (0, 4, 1, 2, 3))
    encoding_indices = idx_blocked.reshape(B, D, H, W)
    return (quantized_st, loss[0, 0], encoding_indices)


# speedup vs baseline: 1.8038x; 1.2616x over previous
import functools

import jax
import jax.numpy as jnp
from jax.experimental import pallas as pl
from jax.experimental.pallas import tpu as pltpu

_NE = 1024
_DIM = 64
_CC = 0.25
_TB = 2048
_NTOK = 32768
_NB = _NTOK // _TB
_NELEM = float(_NTOK * _DIM)


def _vq_body(x_ref, w_ref, qst_ref, idx_ref, loss_ref):
    x = x_ref[...]
    w = w_ref[...]
    x2 = jnp.sum(x * x, axis=1, keepdims=True)

    s = jax.lax.dot_general(
        x, w, (((1,), (1,)), ((), ())),
        preferred_element_type=jnp.float32,
        precision=jax.lax.Precision.DEFAULT)
    w2 = jnp.sum(w * w, axis=1)
    d = (x2 + w2[None, :]) - 2.0 * s
    dmin = jnp.min(d, axis=1, keepdims=True)
    iota = jax.lax.broadcasted_iota(jnp.int32, (_TB, _NE), 1)
    idx = jnp.min(jnp.where(d == dmin, iota, _NE), axis=1).astype(jnp.int32)

    part = jnp.sum(dmin)

    step = pl.program_id(0)

    @pl.when(step == 0)
    def _init():
        loss_ref[0, 0] = 0.0

    loss_ref[0, 0] += part

    @pl.when(step == _NB - 1)
    def _finalize():
        m = loss_ref[0, 0] / _NELEM
        loss_ref[0, 0] = m + _CC * m

    onehot = (iota == idx[:, None]).astype(jnp.bfloat16)
    w_hi = w.astype(jnp.bfloat16)
    w_lo = (w - w_hi.astype(jnp.float32)).astype(jnp.bfloat16)
    q_hi = jax.lax.dot_general(
        onehot, w_hi, (((1,), (0,)), ((), ())),
        preferred_element_type=jnp.float32,
        precision=jax.lax.Precision.DEFAULT)
    q_lo = jax.lax.dot_general(
        onehot, w_lo, (((1,), (0,)), ((), ())),
        preferred_element_type=jnp.float32,
        precision=jax.lax.Precision.DEFAULT)
    q = q_hi + q_lo

    qst_ref[...] = x + (q - x)
    idx_ref[0, 0, :] = idx


@functools.partial(jax.jit, static_argnums=())
def kernel(inputs, weight):
    B, C, D, H, W = inputs.shape
    flat = jnp.transpose(inputs, (0, 2, 3, 4, 1)).reshape(-1, C)

    qst_flat, idx_blocked, loss = pl.pallas_call(
        _vq_body,
        grid=(_NB,),
        in_specs=[
            pl.BlockSpec((_TB, _DIM), lambda i: (i, 0)),
            pl.BlockSpec((_NE, _DIM), lambda i: (0, 0)),
        ],
        out_specs=[
            pl.BlockSpec((_TB, _DIM), lambda i: (i, 0)),
            pl.BlockSpec((1, 1, _TB), lambda i: (i, 0, 0)),
            pl.BlockSpec((1, 1), lambda i: (0, 0), memory_space=pltpu.SMEM),
        ],
        out_shape=[
            jax.ShapeDtypeStruct((_NTOK, _DIM), jnp.float32),
            jax.ShapeDtypeStruct((_NB, 1, _TB), jnp.int32),
            jax.ShapeDtypeStruct((1, 1), jnp.float32),
        ],
    )(flat, weight)

    quantized_st = jnp.transpose(
        qst_flat.reshape(B, D, H, W, C), (0, 4, 1, 2, 3))
    encoding_indices = idx_blocked.reshape(B, D, H, W)
    return (quantized_st, loss[0, 0], encoding_indices)


# speedup vs baseline: 1.8597x; 1.0310x over previous
import functools

import jax
import jax.numpy as jnp
from jax.experimental import pallas as pl
from jax.experimental.pallas import tpu as pltpu

_NE = 1024
_DIM = 64
_CC = 0.25
_TB = 4096
_NTOK = 32768
_NB = _NTOK // _TB
_NELEM = float(_NTOK * _DIM)


def _vq_body(x_ref, w_ref, qst_ref, idx_ref, loss_ref):
    x = x_ref[...]
    w = w_ref[...]
    x2 = jnp.sum(x * x, axis=1, keepdims=True)

    s = jax.lax.dot_general(
        x, w, (((1,), (1,)), ((), ())),
        preferred_element_type=jnp.float32,
        precision=jax.lax.Precision.DEFAULT)
    w2 = jnp.sum(w * w, axis=1)
    d = (x2 + w2[None, :]) - 2.0 * s
    dmin = jnp.min(d, axis=1, keepdims=True)
    iota = jax.lax.broadcasted_iota(jnp.int32, (_TB, _NE), 1)
    idx = jnp.min(jnp.where(d == dmin, iota, _NE), axis=1).astype(jnp.int32)

    part = jnp.sum(dmin)

    step = pl.program_id(0)

    @pl.when(step == 0)
    def _init():
        loss_ref[0, 0] = 0.0

    loss_ref[0, 0] += part

    @pl.when(step == _NB - 1)
    def _finalize():
        m = loss_ref[0, 0] / _NELEM
        loss_ref[0, 0] = m + _CC * m

    onehot = (iota == idx[:, None]).astype(jnp.bfloat16)
    w_hi = w.astype(jnp.bfloat16)
    w_lo = (w - w_hi.astype(jnp.float32)).astype(jnp.bfloat16)
    q_hi = jax.lax.dot_general(
        onehot, w_hi, (((1,), (0,)), ((), ())),
        preferred_element_type=jnp.float32,
        precision=jax.lax.Precision.DEFAULT)
    q_lo = jax.lax.dot_general(
        onehot, w_lo, (((1,), (0,)), ((), ())),
        preferred_element_type=jnp.float32,
        precision=jax.lax.Precision.DEFAULT)
    q = q_hi + q_lo

    qst_ref[...] = x + (q - x)
    idx_ref[0, 0, :] = idx


@functools.partial(jax.jit, static_argnums=())
def kernel(inputs, weight):
    B, C, D, H, W = inputs.shape
    flat = jnp.transpose(inputs, (0, 2, 3, 4, 1)).reshape(-1, C)

    qst_flat, idx_blocked, loss = pl.pallas_call(
        _vq_body,
        grid=(_NB,),
        in_specs=[
            pl.BlockSpec((_TB, _DIM), lambda i: (i, 0)),
            pl.BlockSpec((_NE, _DIM), lambda i: (0, 0)),
        ],
        out_specs=[
            pl.BlockSpec((_TB, _DIM), lambda i: (i, 0)),
            pl.BlockSpec((1, 1, _TB), lambda i: (i, 0, 0)),
            pl.BlockSpec((1, 1), lambda i: (0, 0), memory_space=pltpu.SMEM),
        ],
        out_shape=[
            jax.ShapeDtypeStruct((_NTOK, _DIM), jnp.float32),
            jax.ShapeDtypeStruct((_NB, 1, _TB), jnp.int32),
            jax.ShapeDtypeStruct((1, 1), jnp.float32),
        ],
    )(flat, weight)

    quantized_st = jnp.transpose(
        qst_flat.reshape(B, D, H, W, C), (0, 4, 1, 2, 3))
    encoding_indices = idx_blocked.reshape(B, D, H, W)
    return (quantized_st, loss[0, 0], encoding_indices)
